# Initial kernel scaffold; baseline (speedup 1.0000x reference)
import jax
import jax.numpy as jnp
from jax.experimental import pallas as pl


def kernel(A, B):
    raise NotImplementedError("write your pallas kernel here")


# speedup vs baseline: 1.2454x; 1.2454x over previous
import jax
import jax.numpy as jnp
from jax.experimental import pallas as pl


def _chamfer_block_kernel(n_i, a_ref, b_ref, min_a_ref, min_b_ref):
    i = pl.program_id(1)
    a = a_ref[0]
    bm = b_ref[0]
    a2 = jnp.sum(a * a, axis=1, keepdims=True)
    b2 = jnp.sum(bm * bm, axis=1)[None, :]
    inner = jax.lax.dot_general(
        a, bm, (((1,), (1,)), ((), ())), preferred_element_type=jnp.float32
    )
    d2 = jnp.maximum(a2 + b2 - 2.0 * inner, 0.0)
    min_a_ref[0, 0, :] = jnp.sqrt(jnp.min(d2, axis=1))
    colmin = jnp.min(d2, axis=0)

    @pl.when(i == 0)
    def _init():
        min_b_ref[0, 0, :] = colmin

    @pl.when(i > 0)
    def _acc():
        min_b_ref[0, 0, :] = jnp.minimum(min_b_ref[0, 0, :], colmin)

    @pl.when(i == n_i - 1)
    def _fin():
        min_b_ref[0, 0, :] = jnp.sqrt(min_b_ref[0, 0, :])


def kernel(A, B):
    bt, n, d = A.shape
    m = B.shape[1]
    bi = 512
    n_i = n // bi

    import functools
    min_a, min_b = pl.pallas_call(
        functools.partial(_chamfer_block_kernel, n_i),
        grid=(bt, n_i),
        in_specs=[
            pl.BlockSpec((1, bi, d), lambda b, i: (b, i, 0)),
            pl.BlockSpec((1, m, d), lambda b, i: (b, 0, 0)),
        ],
        out_specs=[
            pl.BlockSpec((1, 1, bi), lambda b, i: (b * n_i + i, 0, 0)),
            pl.BlockSpec((1, 1, m), lambda b, i: (b, 0, 0)),
        ],
        out_shape=[
            jax.ShapeDtypeStruct((bt * n_i, 1, bi), jnp.float32),
            jax.ShapeDtypeStruct((bt, 1, m), jnp.float32),
        ],
    )(A, B)
    min_a = min_a.reshape(bt, n)
    min_b = min_b.reshape(bt, m)
    chamfer = jnp.mean(min_a, axis=1) + jnp.mean(min_b, axis=1)
    return jnp.mean(chamfer) / 12.8
